# baseline (speedup 1.0000x reference)
import jax
import jax.numpy as jnp
from jax import lax
from jax.experimental import pallas as pl
from jax.experimental.pallas import tpu as pltpu
from jax.experimental.pallas import tpu_sc as plsc

NE = 1000000
ED = 64
B = 4096
L = 200

_NC = 2
_NS = 16
_NW = _NC * _NS
_BPW = B // _NW
_BLK = ED * L
_IDXW = _BPW * L
_G0, _G1 = 104, 96
_EDP = 2 * ED


def _emb_body(inp_hbm, table_hbm, out_hbm,
              idx_v, idxp_v, rows_a, rows_b, out_v,
              sem_ga, sem_gb, sem_w):
    c = lax.axis_index("c")
    s = lax.axis_index("s")
    wid = s * _NC + c
    b_base = wid * _BPW
    iota_l = lax.iota(jnp.int32, 16) * L

    pltpu.sync_copy(inp_hbm.at[pl.ds(wid * _IDXW, _IDXW)], idx_v)

    @plsc.parallel_loop(0, _IDXW, 16, unroll=8)
    def _clamp(i):
        v = jnp.clip(idx_v[pl.ds(i, 16)], 0, NE - 1)
        idx_v[pl.ds(i, 16)] = v
        idxp_v[pl.ds(i, 16)] = v >> 1

    def issue_gather(rows_ref, sem, i):
        pltpu.async_copy(
            table_hbm.at[idxp_v.at[pl.ds(i * L, _G0)]],
            rows_ref.at[pl.ds(0, _G0)], sem)
        pltpu.async_copy(
            table_hbm.at[idxp_v.at[pl.ds(i * L + _G0, _G1)]],
            rows_ref.at[pl.ds(_G0, _G1)], sem)

    def wait_gather(rows_ref, sem):
        pltpu.make_async_copy(table_hbm.at[pl.ds(0, L)], rows_ref, sem).wait()

    def transpose(rows_ref, i):
        for lt in tuple(k * 16 for k in range(L // 16)) + (L - 16,):
            pv = (idx_v[pl.ds(i * L + lt, 16)] & 1) * ED
            for j in range(16):
                l = lt + j
                half = pv[j]
                for db in range(ED // 16):
                    vals = rows_ref[l, pl.ds(half + db * 16, 16)]
                    plsc.store_scatter(
                        out_v, [iota_l + (db * 16 * L + l)], vals)

    def issue_write(b):
        pltpu.async_copy(out_v, out_hbm.at[pl.ds(b * _BLK, _BLK)], sem_w)

    def wait_write(b):
        pltpu.make_async_copy(
            out_v, out_hbm.at[pl.ds(b * _BLK, _BLK)], sem_w).wait()

    issue_gather(rows_a, sem_ga, 0)

    def do_pair(p, carry):
        i0 = 2 * p
        b0 = b_base + i0
        issue_gather(rows_b, sem_gb, i0 + 1)
        wait_gather(rows_a, sem_ga)

        @pl.when(p > 0)
        def _():
            wait_write(b0 - 1)

        transpose(rows_a, i0)
        issue_write(b0)

        @pl.when(p < _BPW // 2 - 1)
        def _():
            issue_gather(rows_a, sem_ga, i0 + 2)

        wait_gather(rows_b, sem_gb)
        wait_write(b0)
        transpose(rows_b, i0 + 1)
        issue_write(b0 + 1)
        return carry

    lax.fori_loop(0, _BPW // 2, do_pair, 0)
    wait_write(b_base + _BPW - 1)


@jax.jit
def _emb_encoder(inp, table):
    mesh = plsc.VectorSubcoreMesh(core_axis_name="c", subcore_axis_name="s")
    out = pl.kernel(
        _emb_body,
        mesh=mesh,
        compiler_params=pltpu.CompilerParams(
            needs_layout_passes=False, use_tc_tiling_on_sc=True),
        out_type=jax.ShapeDtypeStruct((B * _BLK,), jnp.float32),
        scratch_types=[
            pltpu.VMEM((_IDXW,), jnp.int32),
            pltpu.VMEM((_IDXW,), jnp.int32),
            pltpu.VMEM((L, _EDP), jnp.float32),
            pltpu.VMEM((L, _EDP), jnp.float32),
            pltpu.VMEM((_BLK,), jnp.float32),
            pltpu.SemaphoreType.DMA,
            pltpu.SemaphoreType.DMA,
            pltpu.SemaphoreType.DMA,
        ],
    )(inp, table)
    return out


def kernel(input, emb_weight):
    inp = input.astype(jnp.int32).reshape(B * L)
    table = emb_weight.reshape(NE // 2, _EDP)
    out = _emb_encoder(inp, table)
    return out.reshape(B, ED, L)


# speedup vs baseline: 1.0776x; 1.0776x over previous
import jax
import jax.numpy as jnp
from jax import lax
from jax.experimental import pallas as pl
from jax.experimental.pallas import tpu as pltpu
from jax.experimental.pallas import tpu_sc as plsc

NE = 1000000
ED = 64
B = 4096
L = 200

_NC = 2
_NS = 16
_NW = _NC * _NS
_BPW = B // _NW
_BLK = ED * L
_IDXW = _BPW * L
_NT = L // 16 + 1
_LR = _NT * 16
_L_OFFS = tuple(k * 16 for k in range(L // 16)) + (L - 16,)


def _emb_body(inp_hbm, table_hbm, out_hbm,
              idx_v, rows_a, rows_b, out_a, out_b,
              sem_ga, sem_gb, sem_wa, sem_wb):
    c = lax.axis_index("c")
    s = lax.axis_index("s")
    wid = s * _NC + c
    b_base = wid * _BPW
    iota_l = lax.iota(jnp.int32, 16) * L

    pltpu.sync_copy(inp_hbm.at[pl.ds(wid * _IDXW, _IDXW)], idx_v)

    @plsc.parallel_loop(0, _IDXW, 16, unroll=8)
    def _clamp(i):
        idx_v[pl.ds(i, 16)] = jnp.clip(idx_v[pl.ds(i, 16)], 0, NE - 1)

    def issue_gather(rows_ref, sem, i):
        for t, off in enumerate(_L_OFFS):
            iv = idx_v[pl.ds(i * L + off, 16)] >> 1
            pltpu.async_copy(
                table_hbm.at[iv], rows_ref.at[pl.ds(t * 16, 16)], sem)

    def wait_gather(rows_ref, sem):
        pltpu.make_async_copy(
            table_hbm.at[pl.ds(0, _LR)], rows_ref, sem).wait()

    def transpose(rows_ref, out_ref, i):
        @plsc.parallel_loop(0, _NT, 1, unroll=2)
        def _t(t):
            l0 = jnp.where(t == _NT - 1, L - 16, t * 16)
            pv = (idx_v[pl.ds(i * L + l0, 16)] & 1) * ED
            for j in range(16):
                half = pv[j]
                for db in range(ED // 16):
                    vals = rows_ref[t * 16 + j, pl.ds(half + db * 16, 16)]
                    plsc.store_scatter(
                        out_ref, [iota_l + (db * 16 * L + l0 + j)], vals)

    def issue_write(out_ref, sem, b):
        pltpu.async_copy(out_ref, out_hbm.at[pl.ds(b * _BLK, _BLK)], sem)

    def wait_write(out_ref, sem, b):
        pltpu.make_async_copy(
            out_ref, out_hbm.at[pl.ds(b * _BLK, _BLK)], sem).wait()

    issue_gather(rows_a, sem_ga, 0)

    def do_pair(p, carry):
        i0 = 2 * p
        b0 = b_base + i0
        issue_gather(rows_b, sem_gb, i0 + 1)
        wait_gather(rows_a, sem_ga)

        @pl.when(p > 0)
        def _():
            wait_write(out_a, sem_wa, b0 - 2)

        transpose(rows_a, out_a, i0)
        issue_write(out_a, sem_wa, b0)

        @pl.when(p < _BPW // 2 - 1)
        def _():
            issue_gather(rows_a, sem_ga, i0 + 2)

        wait_gather(rows_b, sem_gb)

        @pl.when(p > 0)
        def _():
            wait_write(out_b, sem_wb, b0 - 1)

        transpose(rows_b, out_b, i0 + 1)
        issue_write(out_b, sem_wb, b0 + 1)
        return carry

    lax.fori_loop(0, _BPW // 2, do_pair, 0)
    wait_write(out_a, sem_wa, b_base + _BPW - 2)
    wait_write(out_b, sem_wb, b_base + _BPW - 1)


@jax.jit
def _emb_encoder(inp, table):
    mesh = plsc.VectorSubcoreMesh(core_axis_name="c", subcore_axis_name="s")
    out = pl.kernel(
        _emb_body,
        mesh=mesh,
        compiler_params=pltpu.CompilerParams(
            needs_layout_passes=False, use_tc_tiling_on_sc=True),
        out_type=jax.ShapeDtypeStruct((B * _BLK,), jnp.float32),
        scratch_types=[
            pltpu.VMEM((_IDXW,), jnp.int32),
            pltpu.VMEM((_LR, 2 * ED), jnp.float32),
            pltpu.VMEM((_LR, 2 * ED), jnp.float32),
            pltpu.VMEM((_BLK,), jnp.float32),
            pltpu.VMEM((_BLK,), jnp.float32),
            pltpu.SemaphoreType.DMA,
            pltpu.SemaphoreType.DMA,
            pltpu.SemaphoreType.DMA,
            pltpu.SemaphoreType.DMA,
        ],
    )(inp, table)
    return out


def kernel(input, emb_weight):
    inp = input.astype(jnp.int32).reshape(B * L)
    table = emb_weight.reshape(NE // 2, 2 * ED)
    out = _emb_encoder(inp, table)
    return out.reshape(B, ED, L)


# speedup vs baseline: 1.1279x; 1.0467x over previous
import jax
import jax.numpy as jnp
from jax import lax
from jax.experimental import pallas as pl
from jax.experimental.pallas import tpu as pltpu
from jax.experimental.pallas import tpu_sc as plsc

NE = 1000000
ED = 64
B = 4096
L = 200

_NC = 2
_NS = 16
_NW = _NC * _NS
_BPW = B // _NW
_BLK = ED * L
_IDXW = _BPW * L
_NT = L // 16 + 1
_LR = _NT * 16
_L_OFFS = tuple(k * 16 for k in range(L // 16)) + (L - 16,)


def _emb_body(inp_hbm, table_hbm, out_hbm,
              idx_v, rows_a, rows_b, out_a, out_b,
              sem_ga, sem_gb, sem_wa, sem_wb):
    c = lax.axis_index("c")
    s = lax.axis_index("s")
    wid = s * _NC + c
    b_base = wid * _BPW
    iota_l = lax.iota(jnp.int32, 16) * L

    pltpu.sync_copy(inp_hbm.at[pl.ds(wid * _IDXW, _IDXW)], idx_v)

    @plsc.parallel_loop(0, _IDXW, 16, unroll=8)
    def _clamp(i):
        idx_v[pl.ds(i, 16)] = jnp.clip(idx_v[pl.ds(i, 16)], 0, NE - 1)

    def issue_gather(rows_ref, sem, i):
        for t, off in enumerate(_L_OFFS):
            iv = idx_v[pl.ds(i * L + off, 16)]
            pltpu.async_copy(
                table_hbm.at[iv], rows_ref.at[pl.ds(t * 16, 16)], sem)

    def wait_gather(rows_ref, sem):
        pltpu.make_async_copy(
            table_hbm.at[pl.ds(0, _LR)], rows_ref, sem).wait()

    def transpose(rows_ref, out_ref):
        @plsc.parallel_loop(0, _NT, 1, unroll=2)
        def _t(t):
            l0 = jnp.where(t == _NT - 1, L - 16, t * 16)
            for j in range(16):
                for db in range(ED // 16):
                    vals = rows_ref[t * 16 + j, pl.ds(db * 16, 16)]
                    plsc.store_scatter(
                        out_ref, [iota_l + (db * 16 * L + l0 + j)], vals)

    def issue_write(out_ref, sem, b):
        pltpu.async_copy(out_ref, out_hbm.at[pl.ds(b * _BLK, _BLK)], sem)

    def wait_write(out_ref, sem, b):
        pltpu.make_async_copy(
            out_ref, out_hbm.at[pl.ds(b * _BLK, _BLK)], sem).wait()

    issue_gather(rows_a, sem_ga, 0)

    def do_pair(p, carry):
        i0 = 2 * p
        b0 = b_base + i0
        issue_gather(rows_b, sem_gb, i0 + 1)
        wait_gather(rows_a, sem_ga)

        @pl.when(p > 0)
        def _():
            wait_write(out_a, sem_wa, b0 - 2)

        transpose(rows_a, out_a)
        issue_write(out_a, sem_wa, b0)

        @pl.when(p < _BPW // 2 - 1)
        def _():
            issue_gather(rows_a, sem_ga, i0 + 2)

        wait_gather(rows_b, sem_gb)

        @pl.when(p > 0)
        def _():
            wait_write(out_b, sem_wb, b0 - 1)

        transpose(rows_b, out_b)
        issue_write(out_b, sem_wb, b0 + 1)
        return carry

    lax.fori_loop(0, _BPW // 2, do_pair, 0)
    wait_write(out_a, sem_wa, b_base + _BPW - 2)
    wait_write(out_b, sem_wb, b_base + _BPW - 1)


@jax.jit
def _emb_encoder(inp, table):
    mesh = plsc.VectorSubcoreMesh(core_axis_name="c", subcore_axis_name="s")
    out = pl.kernel(
        _emb_body,
        mesh=mesh,
        compiler_params=pltpu.CompilerParams(
            needs_layout_passes=False, use_tc_tiling_on_sc=False),
        out_type=jax.ShapeDtypeStruct((B * _BLK,), jnp.float32),
        scratch_types=[
            pltpu.VMEM((_IDXW,), jnp.int32),
            pltpu.VMEM((_LR, ED), jnp.float32),
            pltpu.VMEM((_LR, ED), jnp.float32),
            pltpu.VMEM((_BLK,), jnp.float32),
            pltpu.VMEM((_BLK,), jnp.float32),
            pltpu.SemaphoreType.DMA,
            pltpu.SemaphoreType.DMA,
            pltpu.SemaphoreType.DMA,
            pltpu.SemaphoreType.DMA,
        ],
    )(inp, table)
    return out


def kernel(input, emb_weight):
    inp = input.astype(jnp.int32).reshape(B * L)
    out = _emb_encoder(inp, emb_weight)
    return out.reshape(B, ED, L)


# speedup vs baseline: 1.3435x; 1.1912x over previous
import jax
import jax.numpy as jnp
from jax import lax
from jax.experimental import pallas as pl
from jax.experimental.pallas import tpu as pltpu
from jax.experimental.pallas import tpu_sc as plsc

NE = 1000000
ED = 64
B = 4096
L = 200

_NC = 2
_NS = 16
_NW = _NC * _NS
_BPW = B // _NW
_BLK = ED * L
_IDXW = _BPW * L
_G0, _G1 = 104, 96


def _emb_body(inp_hbm, table_hbm, out_hbm,
              idx_v, rows_a, rows_b, out_a, out_b,
              sem_ga, sem_gb, sem_wa, sem_wb):
    c = lax.axis_index("c")
    s = lax.axis_index("s")
    wid = s * _NC + c
    b_base = wid * _BPW
    iota_l = lax.iota(jnp.int32, 16) * L

    pltpu.sync_copy(inp_hbm.at[pl.ds(wid * _IDXW, _IDXW)], idx_v)

    @plsc.parallel_loop(0, _IDXW, 16, unroll=8)
    def _clamp(i):
        idx_v[pl.ds(i, 16)] = jnp.clip(idx_v[pl.ds(i, 16)], 0, NE - 1)

    def issue_gather(rows_ref, sem, i):
        pltpu.async_copy(
            table_hbm.at[idx_v.at[pl.ds(i * L, _G0)]],
            rows_ref.at[pl.ds(0, _G0)], sem)
        pltpu.async_copy(
            table_hbm.at[idx_v.at[pl.ds(i * L + _G0, _G1)]],
            rows_ref.at[pl.ds(_G0, _G1)], sem)

    def wait_gather(rows_ref, sem):
        pltpu.make_async_copy(table_hbm.at[pl.ds(0, L)], rows_ref, sem).wait()

    def transpose(rows_ref, out_ref):
        @plsc.parallel_loop(0, L, 1, unroll=8)
        def _t(l):
            for db in range(ED // 16):
                vals = rows_ref[l, pl.ds(db * 16, 16)]
                plsc.store_scatter(
                    out_ref, [iota_l + (db * 16 * L + l)], vals)

    def issue_write(out_ref, sem, b):
        pltpu.async_copy(out_ref, out_hbm.at[pl.ds(b * _BLK, _BLK)], sem)

    def wait_write(out_ref, sem, b):
        pltpu.make_async_copy(
            out_ref, out_hbm.at[pl.ds(b * _BLK, _BLK)], sem).wait()

    issue_gather(rows_a, sem_ga, 0)

    def do_pair(p, carry):
        i0 = 2 * p
        b0 = b_base + i0
        issue_gather(rows_b, sem_gb, i0 + 1)
        wait_gather(rows_a, sem_ga)

        @pl.when(p > 0)
        def _():
            wait_write(out_a, sem_wa, b0 - 2)

        transpose(rows_a, out_a)
        issue_write(out_a, sem_wa, b0)

        @pl.when(p < _BPW // 2 - 1)
        def _():
            issue_gather(rows_a, sem_ga, i0 + 2)

        wait_gather(rows_b, sem_gb)

        @pl.when(p > 0)
        def _():
            wait_write(out_b, sem_wb, b0 - 1)

        transpose(rows_b, out_b)
        issue_write(out_b, sem_wb, b0 + 1)
        return carry

    lax.fori_loop(0, _BPW // 2, do_pair, 0)
    wait_write(out_a, sem_wa, b_base + _BPW - 2)
    wait_write(out_b, sem_wb, b_base + _BPW - 1)


@jax.jit
def _emb_encoder(inp, table):
    mesh = plsc.VectorSubcoreMesh(core_axis_name="c", subcore_axis_name="s")
    out = pl.kernel(
        _emb_body,
        mesh=mesh,
        compiler_params=pltpu.CompilerParams(
            needs_layout_passes=False, use_tc_tiling_on_sc=False),
        out_type=jax.ShapeDtypeStruct((B * _BLK,), jnp.float32),
        scratch_types=[
            pltpu.VMEM((_IDXW,), jnp.int32),
            pltpu.VMEM((L, ED), jnp.float32),
            pltpu.VMEM((L, ED), jnp.float32),
            pltpu.VMEM((_BLK,), jnp.float32),
            pltpu.VMEM((_BLK,), jnp.float32),
            pltpu.SemaphoreType.DMA,
            pltpu.SemaphoreType.DMA,
            pltpu.SemaphoreType.DMA,
            pltpu.SemaphoreType.DMA,
        ],
    )(inp, table)
    return out


def kernel(input, emb_weight):
    inp = input.astype(jnp.int32).reshape(B * L)
    out = _emb_encoder(inp, emb_weight)
    return out.reshape(B, ED, L)


# speedup vs baseline: 1.3460x; 1.0019x over previous
import jax
import jax.numpy as jnp
from jax import lax
from jax.experimental import pallas as pl
from jax.experimental.pallas import tpu as pltpu
from jax.experimental.pallas import tpu_sc as plsc

NE = 1000000
ED = 64
B = 4096
L = 200

_NC = 2
_NS = 16
_NW = _NC * _NS
_BPW = B // _NW
_BLK = ED * L
_IDXW = _BPW * L
_G0, _G1 = 104, 96


def _emb_body(inp_hbm, table_hbm, out_hbm,
              idx_v, rows_a, rows_b, out_a, out_b,
              sem_ga, sem_gb, sem_wa, sem_wb):
    c = lax.axis_index("c")
    s = lax.axis_index("s")
    wid = s * _NC + c
    b_base = wid * _BPW
    iota_l = lax.iota(jnp.int32, 16) * L

    pltpu.sync_copy(inp_hbm.at[pl.ds(wid * _IDXW, 2 * L)],
                    idx_v.at[pl.ds(0, 2 * L)])

    @plsc.parallel_loop(0, 2 * L, 16, unroll=8)
    def _clamp0(i):
        idx_v[pl.ds(i, 16)] = jnp.clip(idx_v[pl.ds(i, 16)], 0, NE - 1)

    def issue_gather(rows_ref, sem, i):
        pltpu.async_copy(
            table_hbm.at[idx_v.at[pl.ds(i * L, _G0)]],
            rows_ref.at[pl.ds(0, _G0)], sem)
        pltpu.async_copy(
            table_hbm.at[idx_v.at[pl.ds(i * L + _G0, _G1)]],
            rows_ref.at[pl.ds(_G0, _G1)], sem)

    def wait_gather(rows_ref, sem):
        pltpu.make_async_copy(table_hbm.at[pl.ds(0, L)], rows_ref, sem).wait()

    def transpose(rows_ref, out_ref):
        @plsc.parallel_loop(0, L, 1, unroll=8)
        def _t(l):
            for db in range(ED // 16):
                vals = rows_ref[l, pl.ds(db * 16, 16)]
                plsc.store_scatter(
                    out_ref, [iota_l + (db * 16 * L + l)], vals)

    def issue_write(out_ref, sem, b):
        pltpu.async_copy(out_ref, out_hbm.at[pl.ds(b * _BLK, _BLK)], sem)

    def wait_write(out_ref, sem, b):
        pltpu.make_async_copy(
            out_ref, out_hbm.at[pl.ds(b * _BLK, _BLK)], sem).wait()

    issue_gather(rows_a, sem_ga, 0)
    issue_gather(rows_b, sem_gb, 1)

    pltpu.sync_copy(inp_hbm.at[pl.ds(wid * _IDXW + 2 * L, _IDXW - 2 * L)],
                    idx_v.at[pl.ds(2 * L, _IDXW - 2 * L)])

    @plsc.parallel_loop(2 * L, _IDXW, 16, unroll=8)
    def _clamp1(i):
        idx_v[pl.ds(i, 16)] = jnp.clip(idx_v[pl.ds(i, 16)], 0, NE - 1)

    def do_pair(p, carry):
        i0 = 2 * p
        b0 = b_base + i0
        wait_gather(rows_a, sem_ga)

        @pl.when(p > 0)
        def _():
            wait_write(out_a, sem_wa, b0 - 2)

        transpose(rows_a, out_a)
        issue_write(out_a, sem_wa, b0)

        @pl.when(p < _BPW // 2 - 1)
        def _():
            issue_gather(rows_a, sem_ga, i0 + 2)

        wait_gather(rows_b, sem_gb)

        @pl.when(p > 0)
        def _():
            wait_write(out_b, sem_wb, b0 - 1)

        transpose(rows_b, out_b)
        issue_write(out_b, sem_wb, b0 + 1)

        @pl.when(p < _BPW // 2 - 1)
        def _():
            issue_gather(rows_b, sem_gb, i0 + 3)

        return carry

    lax.fori_loop(0, _BPW // 2, do_pair, 0)
    wait_write(out_a, sem_wa, b_base + _BPW - 2)
    wait_write(out_b, sem_wb, b_base + _BPW - 1)


@jax.jit
def _emb_encoder(inp, table):
    mesh = plsc.VectorSubcoreMesh(core_axis_name="c", subcore_axis_name="s")
    out = pl.kernel(
        _emb_body,
        mesh=mesh,
        compiler_params=pltpu.CompilerParams(
            needs_layout_passes=False, use_tc_tiling_on_sc=False),
        out_type=jax.ShapeDtypeStruct((B * _BLK,), jnp.float32),
        scratch_types=[
            pltpu.VMEM((_IDXW,), jnp.int32),
            pltpu.VMEM((L, ED), jnp.float32),
            pltpu.VMEM((L, ED), jnp.float32),
            pltpu.VMEM((_BLK,), jnp.float32),
            pltpu.VMEM((_BLK,), jnp.float32),
            pltpu.SemaphoreType.DMA,
            pltpu.SemaphoreType.DMA,
            pltpu.SemaphoreType.DMA,
            pltpu.SemaphoreType.DMA,
        ],
    )(inp, table)
    return out


def kernel(input, emb_weight):
    inp = input.astype(jnp.int32).reshape(B * L)
    out = _emb_encoder(inp, emb_weight)
    return out.reshape(B, ED, L)
